# baseline (speedup 1.0000x reference)
import functools

import jax
import jax.numpy as jnp
from jax import lax
from jax.experimental import pallas as pl
from jax.experimental.pallas import tpu as pltpu
from jax.experimental.pallas import tpu_sc as plsc

NUM_CORES = 2
NUM_SUBCORES = 16
NW = NUM_CORES * NUM_SUBCORES
BC = 128

_mesh = plsc.VectorSubcoreMesh(core_axis_name="core", subcore_axis_name="subcore")
_sc_params = pltpu.CompilerParams(use_tc_tiling_on_sc=False)


def _deg_kernel(npad, ch, rpt):
  @functools.partial(
      pl.kernel,
      out_type=jax.ShapeDtypeStruct((NUM_CORES, npad, 16), jnp.float32),
      mesh=_mesh,
      compiler_params=_sc_params,
      scratch_types=[
          pltpu.VMEM((ch, BC), jnp.int32),
          pltpu.VMEM((BC, 16), jnp.float32),
          pltpu.VMEM_SHARED((npad, 16), jnp.float32),
      ],
  )
  def k(dstp_hbm, ones_hbm, zeros_hbm, out_hbm, dst_v, ones_v, acc):
    c = lax.axis_index("core")
    s = lax.axis_index("subcore")
    w = c * NUM_SUBCORES + s
    pltpu.sync_copy(dstp_hbm.at[w], dst_v)
    pltpu.sync_copy(ones_hbm, ones_v)
    pltpu.sync_copy(zeros_hbm, acc.at[pl.ds(s * rpt, rpt)])
    plsc.subcore_barrier()

    @pl.loop(0, ch)
    def _(j):
      pltpu.sync_copy(ones_v, acc.at[dst_v.at[j]], add=True)

    plsc.subcore_barrier()
    pltpu.sync_copy(acc.at[pl.ds(s * rpt, rpt)],
                    out_hbm.at[c, pl.ds(s * rpt, rpt)])

  return k


def _agg_kernel(npad, ch, rpt):
  @functools.partial(
      pl.kernel,
      out_type=jax.ShapeDtypeStruct((NUM_CORES, npad, 16), jnp.float32),
      mesh=_mesh,
      compiler_params=_sc_params,
      scratch_types=[
          pltpu.VMEM((ch * BC,), jnp.int32),
          pltpu.VMEM((ch * BC,), jnp.int32),
          pltpu.VMEM((ch * BC, 16), jnp.float32),
          pltpu.VMEM_SHARED((npad, 16), jnp.float32),
          pltpu.SemaphoreType.DMA,
      ],
  )
  def k(t_hbm, srcf_hbm, dstf_hbm, zeros_hbm, out_hbm,
        src_v, dst_v, rows, acc, sem0):
    c = lax.axis_index("core")
    s = lax.axis_index("subcore")
    w = c * NUM_SUBCORES + s
    pltpu.sync_copy(srcf_hbm.at[w], src_v)
    pltpu.sync_copy(dstf_hbm.at[w], dst_v)
    pltpu.sync_copy(zeros_hbm, acc.at[pl.ds(s * rpt, rpt)])
    plsc.subcore_barrier()

    pltpu.async_copy(t_hbm.at[src_v], rows, sem0).wait()
    pltpu.sync_copy(rows, acc.at[dst_v], add=True)

    plsc.subcore_barrier()
    pltpu.sync_copy(acc.at[pl.ds(s * rpt, rpt)],
                    out_hbm.at[c, pl.ds(s * rpt, rpt)])

  return k


def kernel(x, edge_index, W1, b1, W2, b2):
  n, d = x.shape
  h = W1.shape[1]
  e = edge_index.shape[1]

  ept = -(-e // (NW * BC)) * BC
  ch = ept // BC
  rpt = -(-(n + 1) // (NUM_SUBCORES * 8)) * 8
  npad = rpt * NUM_SUBCORES

  src = edge_index[0]
  dst = edge_index[1]
  pad = NW * ept - e
  srcp = jnp.concatenate([src, jnp.zeros((pad,), jnp.int32)]).reshape(NW, ch, BC)
  dstp = jnp.concatenate([dst, jnp.full((pad,), n, jnp.int32)]).reshape(NW, ch, BC)
  srcf = srcp.reshape(NW, ch * BC)
  dstf = dstp.reshape(NW, ch * BC)
  zeros16 = jnp.zeros((rpt, 16), jnp.float32)
  ones16 = jnp.ones((BC, 16), jnp.float32)
  b1r = b1.reshape(1, h)
  b2r = b2.reshape(1, W2.shape[1])

  deg_k = _deg_kernel(npad, ch, rpt)
  agg_k = _agg_kernel(npad, ch, rpt)

  bn = 2000
  grid = (n // bn,)

  def _k_mm(x_ref, w_ref, o_ref):
    o_ref[...] = jnp.dot(x_ref[...], w_ref[...],
                         preferred_element_type=jnp.float32)

  hh = pl.pallas_call(
      _k_mm,
      grid=grid,
      in_specs=[pl.BlockSpec((bn, d), lambda i: (i, 0)),
                pl.BlockSpec((d, h), lambda i: (0, 0))],
      out_specs=pl.BlockSpec((bn, h), lambda i: (i, 0)),
      out_shape=jax.ShapeDtypeStruct((n, h), jnp.float32),
  )(x, W1)

  degp = deg_k(dstp, ones16, zeros16)
  p0 = degp[0, :n, 0:1]
  p1 = degp[1, :n, 0:1]

  def _k2(p0_ref, p1_ref, hh_ref, t_ref, dis_ref):
    dis = lax.rsqrt(1.0 + p0_ref[...] + p1_ref[...])
    dis_ref[...] = dis
    t_ref[...] = hh_ref[...] * dis

  t1, dis = pl.pallas_call(
      _k2,
      grid=grid,
      in_specs=[pl.BlockSpec((bn, 1), lambda i: (i, 0)),
                pl.BlockSpec((bn, 1), lambda i: (i, 0)),
                pl.BlockSpec((bn, h), lambda i: (i, 0))],
      out_specs=[pl.BlockSpec((bn, h), lambda i: (i, 0)),
                 pl.BlockSpec((bn, 1), lambda i: (i, 0))],
      out_shape=[jax.ShapeDtypeStruct((n, h), jnp.float32),
                 jax.ShapeDtypeStruct((n, 1), jnp.float32)],
  )(p0, p1, hh)

  s1p = agg_k(t1, srcf, dstf, zeros16)

  def _k4(s0_ref, s1_ref, t_ref, dis_ref, b_ref, u_ref):
    agg = (s0_ref[...] + s1_ref[...] + t_ref[...]) * dis_ref[...] + b_ref[...]
    u_ref[...] = jnp.maximum(agg, 0.0) * dis_ref[...]

  u = pl.pallas_call(
      _k4,
      grid=grid,
      in_specs=[pl.BlockSpec((bn, h), lambda i: (i, 0)),
                pl.BlockSpec((bn, h), lambda i: (i, 0)),
                pl.BlockSpec((bn, h), lambda i: (i, 0)),
                pl.BlockSpec((bn, 1), lambda i: (i, 0)),
                pl.BlockSpec((1, h), lambda i: (0, 0))],
      out_specs=pl.BlockSpec((bn, h), lambda i: (i, 0)),
      out_shape=jax.ShapeDtypeStruct((n, h), jnp.float32),
  )(s1p[0, :n], s1p[1, :n], t1, dis, b1r)

  s2p = agg_k(u, srcf, dstf, zeros16)

  c = W2.shape[1]

  def _k6(s0_ref, s1_ref, u_ref, dis_ref, w_ref, b_ref, o_ref):
    agg = (s0_ref[...] + s1_ref[...] + u_ref[...]) * dis_ref[...]
    o_ref[...] = jnp.dot(agg, w_ref[...],
                         preferred_element_type=jnp.float32) + b_ref[...]

  out = pl.pallas_call(
      _k6,
      grid=grid,
      in_specs=[pl.BlockSpec((bn, h), lambda i: (i, 0)),
                pl.BlockSpec((bn, h), lambda i: (i, 0)),
                pl.BlockSpec((bn, h), lambda i: (i, 0)),
                pl.BlockSpec((bn, 1), lambda i: (i, 0)),
                pl.BlockSpec((h, c), lambda i: (0, 0)),
                pl.BlockSpec((1, c), lambda i: (0, 0))],
      out_specs=pl.BlockSpec((bn, c), lambda i: (i, 0)),
      out_shape=jax.ShapeDtypeStruct((n, c), jnp.float32),
  )(s2p[0, :n], s2p[1, :n], u, dis, W2, b2r)

  return out


# speedup vs baseline: 1.0103x; 1.0103x over previous
import functools

import jax
import jax.numpy as jnp
from jax import lax
from jax.experimental import pallas as pl
from jax.experimental.pallas import tpu as pltpu
from jax.experimental.pallas import tpu_sc as plsc

NUM_CORES = 2
NUM_SUBCORES = 16
NW = NUM_CORES * NUM_SUBCORES
BC = 128

_mesh = plsc.VectorSubcoreMesh(core_axis_name="core", subcore_axis_name="subcore")
_sc_params = pltpu.CompilerParams(use_tc_tiling_on_sc=False)


def _deg_kernel(npad, ch, rpt):
  @functools.partial(
      pl.kernel,
      out_type=jax.ShapeDtypeStruct((NUM_CORES, npad, 16), jnp.float32),
      mesh=_mesh,
      compiler_params=_sc_params,
      scratch_types=[
          pltpu.VMEM((ch, BC), jnp.int32),
          pltpu.VMEM((BC, 16), jnp.float32),
          pltpu.VMEM_SHARED((npad, 16), jnp.float32),
      ],
  )
  def k(dstp_hbm, ones_hbm, zeros_hbm, out_hbm, dst_v, ones_v, acc):
    c = lax.axis_index("core")
    s = lax.axis_index("subcore")
    w = c * NUM_SUBCORES + s
    pltpu.sync_copy(dstp_hbm.at[w], dst_v)
    pltpu.sync_copy(ones_hbm, ones_v)
    pltpu.sync_copy(zeros_hbm, acc.at[pl.ds(s * rpt, rpt)])
    plsc.subcore_barrier()

    @pl.loop(0, ch)
    def _(j):
      pltpu.sync_copy(ones_v, acc.at[dst_v.at[j]], add=True)

    plsc.subcore_barrier()
    pltpu.sync_copy(acc.at[pl.ds(s * rpt, rpt)],
                    out_hbm.at[c, pl.ds(s * rpt, rpt)])

  return k


NPIECE = 4


def _agg_kernel(npad, ch, rpt):
  pp = ch * BC // NPIECE

  @functools.partial(
      pl.kernel,
      out_type=jax.ShapeDtypeStruct((NUM_CORES, npad, 16), jnp.float32),
      mesh=_mesh,
      compiler_params=_sc_params,
      scratch_types=(
          [pltpu.VMEM((pp,), jnp.int32) for _ in range(2 * NPIECE)] + [
              pltpu.VMEM((pp, 16), jnp.float32),
              pltpu.VMEM((pp, 16), jnp.float32),
              pltpu.VMEM_SHARED((npad, 16), jnp.float32),
              pltpu.SemaphoreType.DMA,
              pltpu.SemaphoreType.DMA,
          ]
      ),
  )
  def k(t_hbm, srcf_hbm, dstf_hbm, zeros_hbm, out_hbm, *refs):
    src_vs = refs[:NPIECE]
    dst_vs = refs[NPIECE:2 * NPIECE]
    bufs = refs[2 * NPIECE:2 * NPIECE + 2]
    acc = refs[2 * NPIECE + 2]
    sems = refs[2 * NPIECE + 3:2 * NPIECE + 5]
    c = lax.axis_index("core")
    s = lax.axis_index("subcore")
    w = c * NUM_SUBCORES + s
    for q in range(NPIECE):
      pltpu.sync_copy(srcf_hbm.at[w, q], src_vs[q])
      pltpu.sync_copy(dstf_hbm.at[w, q], dst_vs[q])
    pltpu.sync_copy(zeros_hbm, acc.at[pl.ds(s * rpt, rpt)])
    plsc.subcore_barrier()

    pltpu.async_copy(t_hbm.at[src_vs[0]], bufs[0], sems[0])
    pltpu.async_copy(t_hbm.at[src_vs[1]], bufs[1], sems[1])
    for q in range(NPIECE):
      pltpu.make_async_copy(t_hbm.at[src_vs[q]], bufs[q % 2], sems[q % 2]).wait()
      pltpu.sync_copy(bufs[q % 2], acc.at[dst_vs[q]], add=True)
      if q + 2 < NPIECE:
        pltpu.async_copy(t_hbm.at[src_vs[q + 2]], bufs[q % 2], sems[q % 2])

    plsc.subcore_barrier()
    pltpu.sync_copy(acc.at[pl.ds(s * rpt, rpt)],
                    out_hbm.at[c, pl.ds(s * rpt, rpt)])

  return k


def kernel(x, edge_index, W1, b1, W2, b2):
  n, d = x.shape
  h = W1.shape[1]
  e = edge_index.shape[1]

  ept = -(-e // (NW * BC)) * BC
  ch = ept // BC
  rpt = -(-(n + 1) // (NUM_SUBCORES * 8)) * 8
  npad = rpt * NUM_SUBCORES

  src = edge_index[0]
  dst = edge_index[1]
  pad = NW * ept - e
  srcp = jnp.concatenate([src, jnp.zeros((pad,), jnp.int32)]).reshape(NW, ch, BC)
  dstp = jnp.concatenate([dst, jnp.full((pad,), n, jnp.int32)]).reshape(NW, ch, BC)
  srcf = srcp.reshape(NW, NPIECE, ch * BC // NPIECE)
  dstf = dstp.reshape(NW, NPIECE, ch * BC // NPIECE)
  zeros16 = jnp.zeros((rpt, 16), jnp.float32)
  ones16 = jnp.ones((BC, 16), jnp.float32)
  b1r = b1.reshape(1, h)
  b2r = b2.reshape(1, W2.shape[1])

  deg_k = _deg_kernel(npad, ch, rpt)
  agg_k = _agg_kernel(npad, ch, rpt)

  bn = 2000
  grid = (n // bn,)

  def _k_mm(x_ref, w_ref, o_ref):
    o_ref[...] = jnp.dot(x_ref[...], w_ref[...],
                         preferred_element_type=jnp.float32)

  hh = pl.pallas_call(
      _k_mm,
      grid=grid,
      in_specs=[pl.BlockSpec((bn, d), lambda i: (i, 0)),
                pl.BlockSpec((d, h), lambda i: (0, 0))],
      out_specs=pl.BlockSpec((bn, h), lambda i: (i, 0)),
      out_shape=jax.ShapeDtypeStruct((n, h), jnp.float32),
  )(x, W1)

  degp = deg_k(dstp, ones16, zeros16)
  p0 = degp[0, :n, 0:1]
  p1 = degp[1, :n, 0:1]

  def _k2(p0_ref, p1_ref, hh_ref, t_ref, dis_ref):
    dis = lax.rsqrt(1.0 + p0_ref[...] + p1_ref[...])
    dis_ref[...] = dis
    t_ref[...] = hh_ref[...] * dis

  t1, dis = pl.pallas_call(
      _k2,
      grid=grid,
      in_specs=[pl.BlockSpec((bn, 1), lambda i: (i, 0)),
                pl.BlockSpec((bn, 1), lambda i: (i, 0)),
                pl.BlockSpec((bn, h), lambda i: (i, 0))],
      out_specs=[pl.BlockSpec((bn, h), lambda i: (i, 0)),
                 pl.BlockSpec((bn, 1), lambda i: (i, 0))],
      out_shape=[jax.ShapeDtypeStruct((n, h), jnp.float32),
                 jax.ShapeDtypeStruct((n, 1), jnp.float32)],
  )(p0, p1, hh)

  s1p = agg_k(t1, srcf, dstf, zeros16)

  def _k4(s0_ref, s1_ref, t_ref, dis_ref, b_ref, u_ref):
    agg = (s0_ref[...] + s1_ref[...] + t_ref[...]) * dis_ref[...] + b_ref[...]
    u_ref[...] = jnp.maximum(agg, 0.0) * dis_ref[...]

  u = pl.pallas_call(
      _k4,
      grid=grid,
      in_specs=[pl.BlockSpec((bn, h), lambda i: (i, 0)),
                pl.BlockSpec((bn, h), lambda i: (i, 0)),
                pl.BlockSpec((bn, h), lambda i: (i, 0)),
                pl.BlockSpec((bn, 1), lambda i: (i, 0)),
                pl.BlockSpec((1, h), lambda i: (0, 0))],
      out_specs=pl.BlockSpec((bn, h), lambda i: (i, 0)),
      out_shape=jax.ShapeDtypeStruct((n, h), jnp.float32),
  )(s1p[0, :n], s1p[1, :n], t1, dis, b1r)

  s2p = agg_k(u, srcf, dstf, zeros16)

  c = W2.shape[1]

  def _k6(s0_ref, s1_ref, u_ref, dis_ref, w_ref, b_ref, o_ref):
    agg = (s0_ref[...] + s1_ref[...] + u_ref[...]) * dis_ref[...]
    o_ref[...] = jnp.dot(agg, w_ref[...],
                         preferred_element_type=jnp.float32) + b_ref[...]

  out = pl.pallas_call(
      _k6,
      grid=grid,
      in_specs=[pl.BlockSpec((bn, h), lambda i: (i, 0)),
                pl.BlockSpec((bn, h), lambda i: (i, 0)),
                pl.BlockSpec((bn, h), lambda i: (i, 0)),
                pl.BlockSpec((bn, 1), lambda i: (i, 0)),
                pl.BlockSpec((h, c), lambda i: (0, 0)),
                pl.BlockSpec((1, c), lambda i: (0, 0))],
      out_specs=pl.BlockSpec((bn, c), lambda i: (i, 0)),
      out_shape=jax.ShapeDtypeStruct((n, c), jnp.float32),
  )(s2p[0, :n], s2p[1, :n], u, dis, W2, b2r)

  return out


# speedup vs baseline: 1.0244x; 1.0139x over previous
import functools

import jax
import jax.numpy as jnp
from jax import lax
from jax.experimental import pallas as pl
from jax.experimental.pallas import tpu as pltpu
from jax.experimental.pallas import tpu_sc as plsc

NUM_CORES = 2
NUM_SUBCORES = 16
NW = NUM_CORES * NUM_SUBCORES
BC = 128

_mesh = plsc.VectorSubcoreMesh(core_axis_name="core", subcore_axis_name="subcore")
_sc_params = pltpu.CompilerParams(use_tc_tiling_on_sc=False)


def _deg_kernel(npad, ch, rpt):
  @functools.partial(
      pl.kernel,
      out_type=jax.ShapeDtypeStruct((NUM_CORES, npad, 16), jnp.float32),
      mesh=_mesh,
      compiler_params=_sc_params,
      scratch_types=[
          pltpu.VMEM((ch, BC), jnp.int32),
          pltpu.VMEM((BC, 16), jnp.float32),
          pltpu.VMEM_SHARED((npad, 16), jnp.float32),
      ],
  )
  def k(dstp_hbm, ones_hbm, zeros_hbm, out_hbm, dst_v, ones_v, acc):
    c = lax.axis_index("core")
    s = lax.axis_index("subcore")
    w = c * NUM_SUBCORES + s
    pltpu.sync_copy(dstp_hbm.at[w], dst_v)
    pltpu.sync_copy(ones_hbm, ones_v)
    pltpu.sync_copy(zeros_hbm, acc.at[pl.ds(s * rpt, rpt)])
    plsc.subcore_barrier()

    @pl.loop(0, ch)
    def _(j):
      pltpu.sync_copy(ones_v, acc.at[dst_v.at[j]], add=True)

    plsc.subcore_barrier()
    pltpu.sync_copy(acc.at[pl.ds(s * rpt, rpt)],
                    out_hbm.at[c, pl.ds(s * rpt, rpt)])

  return k


NPIECE = 4


def _agg_kernel(npad, ch, rpt):
  pp = ch * BC // NPIECE

  @functools.partial(
      pl.kernel,
      out_type=jax.ShapeDtypeStruct((NUM_CORES, npad, 16), jnp.float32),
      mesh=_mesh,
      compiler_params=_sc_params,
      scratch_types=(
          [pltpu.VMEM((pp,), jnp.int32) for _ in range(2 * NPIECE)] + [
              pltpu.VMEM((pp, 16), jnp.float32),
              pltpu.VMEM((pp, 16), jnp.float32),
              pltpu.VMEM_SHARED((npad, 16), jnp.float32),
              pltpu.SemaphoreType.DMA,
              pltpu.SemaphoreType.DMA,
          ]
      ),
  )
  def k(t_hbm, srcf_hbm, dstf_hbm, zeros_hbm, out_hbm, *refs):
    src_vs = refs[:NPIECE]
    dst_vs = refs[NPIECE:2 * NPIECE]
    bufs = refs[2 * NPIECE:2 * NPIECE + 2]
    acc = refs[2 * NPIECE + 2]
    sems = refs[2 * NPIECE + 3:2 * NPIECE + 5]
    c = lax.axis_index("core")
    s = lax.axis_index("subcore")
    w = c * NUM_SUBCORES + s
    for q in range(NPIECE):
      pltpu.sync_copy(srcf_hbm.at[w, q], src_vs[q])
      pltpu.sync_copy(dstf_hbm.at[w, q], dst_vs[q])
    pltpu.sync_copy(zeros_hbm, acc.at[pl.ds(s * rpt, rpt)])
    plsc.subcore_barrier()

    pltpu.async_copy(t_hbm.at[src_vs[0]], bufs[0], sems[0])
    pltpu.async_copy(t_hbm.at[src_vs[1]], bufs[1], sems[1])
    for q in range(NPIECE):
      pltpu.make_async_copy(t_hbm.at[src_vs[q]], bufs[q % 2], sems[q % 2]).wait()
      pltpu.sync_copy(bufs[q % 2], acc.at[dst_vs[q]], add=True)
      if q + 2 < NPIECE:
        pltpu.async_copy(t_hbm.at[src_vs[q + 2]], bufs[q % 2], sems[q % 2])

    plsc.subcore_barrier()
    pltpu.sync_copy(acc.at[pl.ds(s * rpt, rpt)],
                    out_hbm.at[c, pl.ds(s * rpt, rpt)])

  return k


def kernel(x, edge_index, W1, b1, W2, b2):
  n, d = x.shape
  h = W1.shape[1]
  e = edge_index.shape[1]

  ept = -(-e // (NW * BC)) * BC
  ch = ept // BC
  rpt = -(-(n + 1) // (NUM_SUBCORES * 64)) * 64
  npad = rpt * NUM_SUBCORES
  nr = npad // 8

  src = edge_index[0]
  dst = edge_index[1]
  pad = NW * ept - e
  srcp = jnp.concatenate([src, jnp.zeros((pad,), jnp.int32)]).reshape(NW, ch, BC)
  dstp = jnp.concatenate([dst, jnp.full((pad,), n, jnp.int32)]).reshape(NW, ch, BC)
  srcf = srcp.reshape(NW, NPIECE, ch * BC // NPIECE)
  dstf = dstp.reshape(NW, NPIECE, ch * BC // NPIECE)
  zeros16 = jnp.zeros((rpt, 16), jnp.float32)
  ones16 = jnp.ones((BC, 16), jnp.float32)
  c = W2.shape[1]
  b1blk = jnp.tile(b1, 8).reshape(1, 8 * h)
  w2blk = jnp.kron(jnp.eye(8, dtype=jnp.float32), W2)
  b2blk = jnp.tile(b2, 8).reshape(1, 8 * c)

  deg_k = _deg_kernel(npad, ch, rpt)
  agg_k = _agg_kernel(npad, ch, rpt)


  bnm = 2048
  gm = (npad // bnm,)

  def _k_mm(x_ref, w_ref, o_ref):
    o_ref[...] = jnp.dot(x_ref[...], w_ref[...],
                         preferred_element_type=jnp.float32)

  hh = pl.pallas_call(
      _k_mm,
      grid=gm,
      in_specs=[pl.BlockSpec((bnm, d), lambda i: (i, 0)),
                pl.BlockSpec((d, h), lambda i: (0, 0))],
      out_specs=pl.BlockSpec((bnm, h), lambda i: (i, 0)),
      out_shape=jax.ShapeDtypeStruct((npad, h), jnp.float32),
  )(x, W1)
  hh128 = hh.reshape(nr, 8 * h)

  degp = deg_k(dstp, ones16, zeros16)
  p0g = degp[0].reshape(nr, 128)
  p1g = degp[1].reshape(nr, 128)

  bne = 256
  ge = (nr // bne,)
  eb = lambda: pl.BlockSpec((bne, 128), lambda i: (i, 0))

  def _k2(p0_ref, p1_ref, hh_ref, t_ref, dis_ref):
    dis = lax.rsqrt(1.0 + p0_ref[...] + p1_ref[...])
    dis_ref[...] = dis
    t_ref[...] = hh_ref[...] * dis

  t128, dis128 = pl.pallas_call(
      _k2,
      grid=ge,
      in_specs=[eb(), eb(), eb()],
      out_specs=[eb(), eb()],
      out_shape=[jax.ShapeDtypeStruct((nr, 128), jnp.float32),
                 jax.ShapeDtypeStruct((nr, 128), jnp.float32)],
  )(p0g, p1g, hh128)

  s1p = agg_k(t128.reshape(npad, 16), srcf, dstf, zeros16)

  def _k4(s0_ref, s1_ref, t_ref, dis_ref, b_ref, u_ref):
    agg = (s0_ref[...] + s1_ref[...] + t_ref[...]) * dis_ref[...] + b_ref[...]
    u_ref[...] = jnp.maximum(agg, 0.0) * dis_ref[...]

  u128 = pl.pallas_call(
      _k4,
      grid=ge,
      in_specs=[eb(), eb(), eb(), eb(),
                pl.BlockSpec((1, 128), lambda i: (0, 0))],
      out_specs=eb(),
      out_shape=jax.ShapeDtypeStruct((nr, 128), jnp.float32),
  )(s1p[0].reshape(nr, 128), s1p[1].reshape(nr, 128), t128, dis128, b1blk)

  s2p = agg_k(u128.reshape(npad, 16), srcf, dstf, zeros16)

  def _k6(s0_ref, s1_ref, u_ref, dis_ref, w_ref, b_ref, o_ref):
    agg = (s0_ref[...] + s1_ref[...] + u_ref[...]) * dis_ref[...]
    o_ref[...] = jnp.dot(agg, w_ref[...],
                         preferred_element_type=jnp.float32) + b_ref[...]

  og = pl.pallas_call(
      _k6,
      grid=ge,
      in_specs=[eb(), eb(), eb(), eb(),
                pl.BlockSpec((128, 8 * c), lambda i: (0, 0)),
                pl.BlockSpec((1, 8 * c), lambda i: (0, 0))],
      out_specs=pl.BlockSpec((bne, 8 * c), lambda i: (i, 0)),
      out_shape=jax.ShapeDtypeStruct((nr, 8 * c), jnp.float32),
  )(s2p[0].reshape(nr, 128), s2p[1].reshape(nr, 128), u128, dis128,
    w2blk, b2blk)

  return og[:n // 8].reshape(n, c)


# speedup vs baseline: 2.1733x; 2.1217x over previous
import functools

import jax
import jax.numpy as jnp
from jax import lax
from jax.experimental import pallas as pl
from jax.experimental.pallas import tpu as pltpu
from jax.experimental.pallas import tpu_sc as plsc

NUM_CORES = 2
NUM_SUBCORES = 16
NW = NUM_CORES * NUM_SUBCORES
NPIECE = 5

_mesh = plsc.VectorSubcoreMesh(core_axis_name="core", subcore_axis_name="subcore")
_sc_params = pltpu.CompilerParams(use_tc_tiling_on_sc=False)


def _deg_kernel(n, e, npad, rpt):
  ept = e // NW
  pp = ept // NPIECE
  nr = npad // 8
  rg = rpt // 8

  @functools.partial(
      pl.kernel,
      out_type=[jax.ShapeDtypeStruct((npad, 16), jnp.float32),
                jax.ShapeDtypeStruct((npad, 16), jnp.float32)],
      mesh=_mesh,
      compiler_params=_sc_params,
      scratch_types=(
          [pltpu.VMEM((pp,), jnp.int32) for _ in range(NPIECE)] + [
              pltpu.VMEM((pp, 16), jnp.float32),
              pltpu.VMEM_SHARED((npad, 16), jnp.float32),
          ]
      ),
  )
  def k(ei_hbm, ones_hbm, zeros_hbm, out0, out1, *refs):
    dst_vs = refs[:NPIECE]
    ones_v = refs[NPIECE]
    acc = refs[NPIECE + 1]
    c = lax.axis_index("core")
    s = lax.axis_index("subcore")
    w = c * NUM_SUBCORES + s
    base = w * ept
    for q in range(NPIECE):
      pltpu.sync_copy(ei_hbm.at[1, pl.ds(base + q * pp, pp)], dst_vs[q])
    pltpu.sync_copy(ones_hbm, ones_v)
    pltpu.sync_copy(zeros_hbm, acc.at[pl.ds(s * rpt, rpt)])
    plsc.subcore_barrier()

    for q in range(NPIECE):
      pltpu.sync_copy(ones_v, acc.at[dst_vs[q]], add=True)

    plsc.subcore_barrier()
    src = acc.at[pl.ds(s * rpt, rpt)]

    @pl.when(c == 0)
    def _():
      pltpu.sync_copy(src, out0.at[pl.ds(s * rpt, rpt)])

    @pl.when(c == 1)
    def _():
      pltpu.sync_copy(src, out1.at[pl.ds(s * rpt, rpt)])

  return k


def _agg_kernel(n, e, npad, rpt):
  ept = e // NW
  pp = ept // NPIECE
  nr = npad // 8
  rg = rpt // 8

  @functools.partial(
      pl.kernel,
      out_type=[jax.ShapeDtypeStruct((npad, 16), jnp.float32),
                jax.ShapeDtypeStruct((npad, 16), jnp.float32)],
      mesh=_mesh,
      compiler_params=_sc_params,
      scratch_types=(
          [pltpu.VMEM((pp,), jnp.int32) for _ in range(2 * NPIECE)] + [
              pltpu.VMEM((pp, 16), jnp.float32),
              pltpu.VMEM((pp, 16), jnp.float32),
              pltpu.VMEM_SHARED((npad, 16), jnp.float32),
              pltpu.SemaphoreType.DMA,
              pltpu.SemaphoreType.DMA,
          ]
      ),
  )
  def k(t_hbm, ei_hbm, zeros_hbm, out0, out1, *refs):
    src_vs = refs[:NPIECE]
    dst_vs = refs[NPIECE:2 * NPIECE]
    bufs = refs[2 * NPIECE:2 * NPIECE + 2]
    acc = refs[2 * NPIECE + 2]
    sems = refs[2 * NPIECE + 3:2 * NPIECE + 5]
    t_tab = t_hbm
    c = lax.axis_index("core")
    s = lax.axis_index("subcore")
    w = c * NUM_SUBCORES + s
    base = w * ept
    for q in range(NPIECE):
      pltpu.sync_copy(ei_hbm.at[0, pl.ds(base + q * pp, pp)], src_vs[q])
      pltpu.sync_copy(ei_hbm.at[1, pl.ds(base + q * pp, pp)], dst_vs[q])
    pltpu.sync_copy(zeros_hbm, acc.at[pl.ds(s * rpt, rpt)])
    plsc.subcore_barrier()

    pltpu.async_copy(t_tab.at[src_vs[0]], bufs[0], sems[0])
    pltpu.async_copy(t_tab.at[src_vs[1]], bufs[1], sems[1])
    for q in range(NPIECE):
      pltpu.make_async_copy(t_tab.at[src_vs[q]], bufs[q % 2], sems[q % 2]).wait()
      pltpu.sync_copy(bufs[q % 2], acc.at[dst_vs[q]], add=True)
      if q + 2 < NPIECE:
        pltpu.async_copy(t_tab.at[src_vs[q + 2]], bufs[q % 2], sems[q % 2])

    plsc.subcore_barrier()
    src = acc.at[pl.ds(s * rpt, rpt)]

    @pl.when(c == 0)
    def _():
      pltpu.sync_copy(src, out0.at[pl.ds(s * rpt, rpt)])

    @pl.when(c == 1)
    def _():
      pltpu.sync_copy(src, out1.at[pl.ds(s * rpt, rpt)])

  return k


def kernel(x, edge_index, W1, b1, W2, b2):
  n, d = x.shape
  h = W1.shape[1]
  e = edge_index.shape[1]
  c = W2.shape[1]

  rpt = -(-(n + 1) // (NUM_SUBCORES * 64)) * 64
  npad = rpt * NUM_SUBCORES
  nr = npad // 8

  zeros16 = jnp.zeros((rpt, 16), jnp.float32)
  ones16 = jnp.ones((e // NW // NPIECE, 16), jnp.float32)
  b1blk = jnp.tile(b1, 8).reshape(1, 8 * h)
  w2blk = jnp.kron(jnp.eye(8, dtype=jnp.float32), W2)
  b2blk = jnp.tile(b2, 8).reshape(1, 8 * c)

  deg_k = _deg_kernel(n, e, npad, rpt)
  agg_k = _agg_kernel(n, e, npad, rpt)

  bnm = 2048

  def _k_mm(x_ref, w_ref, o_ref):
    o_ref[...] = jnp.dot(x_ref[...], w_ref[...],
                         preferred_element_type=jnp.float32)

  hh = pl.pallas_call(
      _k_mm,
      grid=(npad // bnm,),
      in_specs=[pl.BlockSpec((bnm, d), lambda i: (i, 0)),
                pl.BlockSpec((d, h), lambda i: (0, 0))],
      out_specs=pl.BlockSpec((bnm, h), lambda i: (i, 0)),
      out_shape=jax.ShapeDtypeStruct((npad, h), jnp.float32),
  )(x, W1)
  hh128 = hh.reshape(nr, 8 * h)

  p0n, p1n = deg_k(edge_index, ones16, zeros16)
  p0g = p0n.reshape(nr, 128)
  p1g = p1n.reshape(nr, 128)

  bne = 256
  ge = (nr // bne,)
  eb = lambda: pl.BlockSpec((bne, 128), lambda i: (i, 0))

  def _k2(p0_ref, p1_ref, hh_ref, t_ref, dis_ref):
    dis = lax.rsqrt(1.0 + p0_ref[...] + p1_ref[...])
    dis_ref[...] = dis
    t_ref[...] = hh_ref[...] * dis

  t128, dis128 = pl.pallas_call(
      _k2,
      grid=ge,
      in_specs=[eb(), eb(), eb()],
      out_specs=[eb(), eb()],
      out_shape=[jax.ShapeDtypeStruct((nr, 128), jnp.float32),
                 jax.ShapeDtypeStruct((nr, 128), jnp.float32)],
  )(p0g, p1g, hh128)

  s10n, s11n = agg_k(t128.reshape(npad, 16), edge_index, zeros16)
  s10 = s10n.reshape(nr, 128)
  s11 = s11n.reshape(nr, 128)

  def _k4(s0_ref, s1_ref, t_ref, dis_ref, b_ref, u_ref):
    agg = (s0_ref[...] + s1_ref[...] + t_ref[...]) * dis_ref[...] + b_ref[...]
    u_ref[...] = jnp.maximum(agg, 0.0) * dis_ref[...]

  u128 = pl.pallas_call(
      _k4,
      grid=ge,
      in_specs=[eb(), eb(), eb(), eb(),
                pl.BlockSpec((1, 128), lambda i: (0, 0))],
      out_specs=eb(),
      out_shape=jax.ShapeDtypeStruct((nr, 128), jnp.float32),
  )(s10, s11, t128, dis128, b1blk)

  s20n, s21n = agg_k(u128.reshape(npad, 16), edge_index, zeros16)
  s20 = s20n.reshape(nr, 128)
  s21 = s21n.reshape(nr, 128)

  def _k6(s0_ref, s1_ref, u_ref, dis_ref, w_ref, b_ref, o_ref):
    agg = (s0_ref[...] + s1_ref[...] + u_ref[...]) * dis_ref[...]
    o_ref[...] = jnp.dot(agg, w_ref[...],
                         preferred_element_type=jnp.float32) + b_ref[...]

  og = pl.pallas_call(
      _k6,
      grid=ge,
      in_specs=[eb(), eb(), eb(), eb(),
                pl.BlockSpec((128, 8 * c), lambda i: (0, 0)),
                pl.BlockSpec((1, 8 * c), lambda i: (0, 0))],
      out_specs=pl.BlockSpec((bne, 8 * c), lambda i: (i, 0)),
      out_shape=jax.ShapeDtypeStruct((n // 8, 8 * c), jnp.float32),
  )(s20, s21, u128, dis128, w2blk, b2blk)

  return og.reshape(n, c)


# speedup vs baseline: 2.1781x; 1.0022x over previous
import functools

import jax
import jax.numpy as jnp
from jax import lax
from jax.experimental import pallas as pl
from jax.experimental.pallas import tpu as pltpu
from jax.experimental.pallas import tpu_sc as plsc

NUM_CORES = 2
NUM_SUBCORES = 16
NW = NUM_CORES * NUM_SUBCORES
NPIECE = 5

_mesh = plsc.VectorSubcoreMesh(core_axis_name="core", subcore_axis_name="subcore")
_sc_params = pltpu.CompilerParams(use_tc_tiling_on_sc=False)


def _deg_kernel(n, e, npad, rpt):
  ept = e // NW
  pp = ept // NPIECE
  nr = npad // 8
  rg = rpt // 8

  @functools.partial(
      pl.kernel,
      out_type=[jax.ShapeDtypeStruct((npad, 16), jnp.float32),
                jax.ShapeDtypeStruct((npad, 16), jnp.float32)],
      mesh=_mesh,
      compiler_params=_sc_params,
      scratch_types=(
          [pltpu.VMEM((pp,), jnp.int32) for _ in range(NPIECE)] + [
              pltpu.VMEM((pp, 16), jnp.float32),
              pltpu.VMEM_SHARED((npad, 16), jnp.float32),
          ] + [pltpu.SemaphoreType.DMA for _ in range(NPIECE)]
      ),
  )
  def k(ei_hbm, ones_hbm, zeros_hbm, out0, out1, *refs):
    dst_vs = refs[:NPIECE]
    ones_v = refs[NPIECE]
    acc = refs[NPIECE + 1]
    sems = refs[NPIECE + 2:NPIECE + 2 + NPIECE]
    c = lax.axis_index("core")
    s = lax.axis_index("subcore")
    w = c * NUM_SUBCORES + s
    base = w * ept
    for q in range(NPIECE):
      pltpu.sync_copy(ei_hbm.at[1, pl.ds(base + q * pp, pp)], dst_vs[q])
    pltpu.sync_copy(ones_hbm, ones_v)
    pltpu.sync_copy(zeros_hbm, acc.at[pl.ds(s * rpt, rpt)])
    plsc.subcore_barrier()

    descs = [pltpu.async_copy(ones_v, acc.at[dst_vs[q]], sems[q], add=True)
             for q in range(NPIECE)]
    for d_ in descs:
      d_.wait()

    plsc.subcore_barrier()
    src = acc.at[pl.ds(s * rpt, rpt)]

    @pl.when(c == 0)
    def _():
      pltpu.sync_copy(src, out0.at[pl.ds(s * rpt, rpt)])

    @pl.when(c == 1)
    def _():
      pltpu.sync_copy(src, out1.at[pl.ds(s * rpt, rpt)])

  return k


def _agg_kernel(n, e, npad, rpt):
  ept = e // NW
  pp = ept // NPIECE
  nr = npad // 8
  rg = rpt // 8

  @functools.partial(
      pl.kernel,
      out_type=[jax.ShapeDtypeStruct((npad, 16), jnp.float32),
                jax.ShapeDtypeStruct((npad, 16), jnp.float32)],
      mesh=_mesh,
      compiler_params=_sc_params,
      scratch_types=(
          [pltpu.VMEM((pp,), jnp.int32) for _ in range(2 * NPIECE)] + [
              pltpu.VMEM((pp, 16), jnp.float32),
              pltpu.VMEM((pp, 16), jnp.float32),
              pltpu.VMEM_SHARED((npad, 16), jnp.float32),
              pltpu.SemaphoreType.DMA,
              pltpu.SemaphoreType.DMA,
          ]
      ),
  )
  def k(t_hbm, ei_hbm, zeros_hbm, out0, out1, *refs):
    src_vs = refs[:NPIECE]
    dst_vs = refs[NPIECE:2 * NPIECE]
    bufs = refs[2 * NPIECE:2 * NPIECE + 2]
    acc = refs[2 * NPIECE + 2]
    sems = refs[2 * NPIECE + 3:2 * NPIECE + 5]
    t_tab = t_hbm
    c = lax.axis_index("core")
    s = lax.axis_index("subcore")
    w = c * NUM_SUBCORES + s
    base = w * ept
    for q in range(NPIECE):
      pltpu.sync_copy(ei_hbm.at[0, pl.ds(base + q * pp, pp)], src_vs[q])
      pltpu.sync_copy(ei_hbm.at[1, pl.ds(base + q * pp, pp)], dst_vs[q])
    pltpu.sync_copy(zeros_hbm, acc.at[pl.ds(s * rpt, rpt)])
    plsc.subcore_barrier()

    pltpu.async_copy(t_tab.at[src_vs[0]], bufs[0], sems[0])
    pltpu.async_copy(t_tab.at[src_vs[1]], bufs[1], sems[1])
    for q in range(NPIECE):
      pltpu.make_async_copy(t_tab.at[src_vs[q]], bufs[q % 2], sems[q % 2]).wait()
      pltpu.sync_copy(bufs[q % 2], acc.at[dst_vs[q]], add=True)
      if q + 2 < NPIECE:
        pltpu.async_copy(t_tab.at[src_vs[q + 2]], bufs[q % 2], sems[q % 2])

    plsc.subcore_barrier()
    src = acc.at[pl.ds(s * rpt, rpt)]

    @pl.when(c == 0)
    def _():
      pltpu.sync_copy(src, out0.at[pl.ds(s * rpt, rpt)])

    @pl.when(c == 1)
    def _():
      pltpu.sync_copy(src, out1.at[pl.ds(s * rpt, rpt)])

  return k


def kernel(x, edge_index, W1, b1, W2, b2):
  n, d = x.shape
  h = W1.shape[1]
  e = edge_index.shape[1]
  c = W2.shape[1]

  rpt = -(-(n + 1) // (NUM_SUBCORES * 64)) * 64
  npad = rpt * NUM_SUBCORES
  nr = npad // 8

  zeros16 = jnp.zeros((rpt, 16), jnp.float32)
  ones16 = jnp.ones((e // NW // NPIECE, 16), jnp.float32)
  b1blk = jnp.tile(b1, 8).reshape(1, 8 * h)
  w2blk = jnp.kron(jnp.eye(8, dtype=jnp.float32), W2)
  b2blk = jnp.tile(b2, 8).reshape(1, 8 * c)

  deg_k = _deg_kernel(n, e, npad, rpt)
  agg_k = _agg_kernel(n, e, npad, rpt)

  bnm = 2048

  def _k_mm(x_ref, w_ref, o_ref):
    o_ref[...] = jnp.dot(x_ref[...], w_ref[...],
                         preferred_element_type=jnp.float32)

  hh = pl.pallas_call(
      _k_mm,
      grid=(npad // bnm,),
      in_specs=[pl.BlockSpec((bnm, d), lambda i: (i, 0)),
                pl.BlockSpec((d, h), lambda i: (0, 0))],
      out_specs=pl.BlockSpec((bnm, h), lambda i: (i, 0)),
      out_shape=jax.ShapeDtypeStruct((npad, h), jnp.float32),
  )(x, W1)
  hh128 = hh.reshape(nr, 8 * h)

  p0n, p1n = deg_k(edge_index, ones16, zeros16)
  p0g = p0n.reshape(nr, 128)
  p1g = p1n.reshape(nr, 128)

  bne = 256
  ge = (nr // bne,)
  eb = lambda: pl.BlockSpec((bne, 128), lambda i: (i, 0))

  def _k2(p0_ref, p1_ref, hh_ref, t_ref, dis_ref):
    dis = lax.rsqrt(1.0 + p0_ref[...] + p1_ref[...])
    dis_ref[...] = dis
    t_ref[...] = hh_ref[...] * dis

  t128, dis128 = pl.pallas_call(
      _k2,
      grid=ge,
      in_specs=[eb(), eb(), eb()],
      out_specs=[eb(), eb()],
      out_shape=[jax.ShapeDtypeStruct((nr, 128), jnp.float32),
                 jax.ShapeDtypeStruct((nr, 128), jnp.float32)],
  )(p0g, p1g, hh128)

  s10n, s11n = agg_k(t128.reshape(npad, 16), edge_index, zeros16)
  s10 = s10n.reshape(nr, 128)
  s11 = s11n.reshape(nr, 128)

  def _k4(s0_ref, s1_ref, t_ref, dis_ref, b_ref, u_ref):
    agg = (s0_ref[...] + s1_ref[...] + t_ref[...]) * dis_ref[...] + b_ref[...]
    u_ref[...] = jnp.maximum(agg, 0.0) * dis_ref[...]

  u128 = pl.pallas_call(
      _k4,
      grid=ge,
      in_specs=[eb(), eb(), eb(), eb(),
                pl.BlockSpec((1, 128), lambda i: (0, 0))],
      out_specs=eb(),
      out_shape=jax.ShapeDtypeStruct((nr, 128), jnp.float32),
  )(s10, s11, t128, dis128, b1blk)

  s20n, s21n = agg_k(u128.reshape(npad, 16), edge_index, zeros16)
  s20 = s20n.reshape(nr, 128)
  s21 = s21n.reshape(nr, 128)

  def _k6(s0_ref, s1_ref, u_ref, dis_ref, w_ref, b_ref, o_ref):
    agg = (s0_ref[...] + s1_ref[...] + u_ref[...]) * dis_ref[...]
    o_ref[...] = jnp.dot(agg, w_ref[...],
                         preferred_element_type=jnp.float32) + b_ref[...]

  og = pl.pallas_call(
      _k6,
      grid=ge,
      in_specs=[eb(), eb(), eb(), eb(),
                pl.BlockSpec((128, 8 * c), lambda i: (0, 0)),
                pl.BlockSpec((1, 8 * c), lambda i: (0, 0))],
      out_specs=pl.BlockSpec((bne, 8 * c), lambda i: (i, 0)),
      out_shape=jax.ShapeDtypeStruct((n // 8, 8 * c), jnp.float32),
  )(s20, s21, u128, dis128, w2blk, b2blk)

  return og.reshape(n, c)
